# baseline (speedup 1.0000x reference)
import functools

import jax
import jax.numpy as jnp
from jax import lax
from jax.experimental import pallas as pl
from jax.experimental.pallas import tpu as pltpu
from jax.experimental.pallas import tpu_sc as plsc

NUSERS = 10000
NENT = 10000
CH = 128
NEDGE = 320000
NFACT = 4
NRELM1 = 8
NNZ_PAD = 204800

NC = 2
NS = 16
NW = NC * NS
L = 16

CEG = 128
ECHUNK = 128
EPW = 10112
EPAD = EPW * NW
NCH_E = EPW // ECHUNK
CEU = 128
IPW = 6272
NNZ_PAD2 = IPW * NW
NCH_I = IPW // CEU
RPT = NENT // NS
WID = CH + L
SEG = CH // L
NTAB = NRELM1 * NENT
ACC_E = 10016
RPTE = ACC_E // NS
TAILR = NENT - 15 * RPTE

@functools.cache
def _sc_calls():
    mesh = plsc.VectorSubcoreMesh(core_axis_name="c", subcore_axis_name="s",
                                  num_cores=NC, num_subcores=NS)
    sems = [pltpu.SemaphoreType.DMA] * 7
    idx_scratch = [
        pltpu.VMEM((4, ECHUNK), jnp.int32),
        pltpu.VMEM((4, ECHUNK), jnp.int32),
        pltpu.VMEM((4, NJ, CEG), jnp.int32),
        pltpu.VMEM((4, NJ, CEG), jnp.int32),
        pltpu.VMEM((2, ECHUNK, CH), jnp.float32),
    ]
    edge_call_cnt = functools.partial(
        pl.kernel,
        out_type=[
            jax.ShapeDtypeStruct((NC, NENT, CH), jnp.float32),
            jax.ShapeDtypeStruct((NC, NENT, L), jnp.float32),
        ],
        mesh=mesh,
        compiler_params=pltpu.CompilerParams(use_tc_tiling_on_sc=False, needs_layout_passes=False),
        scratch_types=idx_scratch + [pltpu.VMEM((CEG, L), jnp.float32)]
        + sems + [
            pltpu.VMEM_SHARED((ACC_E, CH), jnp.float32),
            pltpu.VMEM_SHARED((ACC_E, L), jnp.float32),
        ],
    )(_edge_body_cnt)
    edge_call_nc = functools.partial(
        pl.kernel,
        out_type=jax.ShapeDtypeStruct((NC, NENT, CH), jnp.float32),
        mesh=mesh,
        compiler_params=pltpu.CompilerParams(use_tc_tiling_on_sc=False, needs_layout_passes=False),
        scratch_types=idx_scratch + sems
        + [pltpu.VMEM_SHARED((ACC_E, CH), jnp.float32)],
    )(_edge_body_nc)
    user_call = functools.partial(
        pl.kernel,
        out_type=jax.ShapeDtypeStruct((NC, NUSERS, CH), jnp.float32),
        mesh=mesh,
        compiler_params=pltpu.CompilerParams(use_tc_tiling_on_sc=False, needs_layout_passes=False),
        scratch_types=[
            pltpu.VMEM((4, CEU), jnp.int32),
            pltpu.VMEM((4, CEU), jnp.int32),
            pltpu.VMEM((4, CEU), jnp.float32),
            pltpu.VMEM((2, CEU, CH), jnp.float32),
            pltpu.SemaphoreType.DMA,
            pltpu.SemaphoreType.DMA,
            pltpu.SemaphoreType.DMA,
            pltpu.SemaphoreType.DMA,
            pltpu.SemaphoreType.DMA,
            pltpu.SemaphoreType.DMA,
            pltpu.SemaphoreType.DMA,
            pltpu.VMEM_SHARED((NUSERS, CH), jnp.float32),
        ],
    )(_user_body)
    return edge_call_cnt, edge_call_nc, user_call


NJ = ECHUNK // CEG


def _edge_body_cnt(tab, tailh, headh, eth, zh, zc, ea, cnt,
                   tailv, etv, headv, fidx, g, onesb,
                   i0, i1, i2, i3, gs, s0, s1, acc, cacc):
    _edge_impl(tab, tailh, headh, eth, zh, ea, tailv, etv, headv, fidx, g,
               i0, i1, i2, i3, gs, s0, s1, acc,
               zc=zc, cnt_out=cnt, onesb=onesb, cacc=cacc)


def _edge_body_nc(tab, tailh, headh, eth, zh, ea,
                  tailv, etv, headv, fidx, g,
                  i0, i1, i2, i3, gs, s0, s1, acc):
    _edge_impl(tab, tailh, headh, eth, zh, ea, tailv, etv, headv, fidx, g,
               i0, i1, i2, i3, gs, s0, s1, acc)


def _edge_impl(tab, tailh, headh, eth, zh, ea_out,
               tailv, etv, headv, fidx, g,
               isem0, isem1, isem2, isem3, gsem, ssem0, ssem1, acc,
               zc=None, cnt_out=None, onesb=None, cacc=None):
    with_cnt = cacc is not None
    cid = lax.axis_index("c")
    sid = lax.axis_index("s")
    w = cid * NS + sid
    r0 = sid * RPTE
    isem = (isem0, isem1, isem2, isem3)
    ssem = (ssem0, ssem1)

    pltpu.sync_copy(zh, acc.at[pl.ds(r0, RPTE)])
    if with_cnt:
        pltpu.sync_copy(zc, cacc.at[pl.ds(r0, RPTE)])
        one = jnp.ones((L,), jnp.float32)
        for e in range(CEG):
            onesb[e, :] = one

    def start_idx(k, b):
        base = w * EPW + k * ECHUNK
        pltpu.async_copy(tailh.at[pl.ds(base, ECHUNK)], tailv.at[b], isem[b])
        pltpu.async_copy(eth.at[pl.ds(base, ECHUNK)], etv.at[b], isem[b])
        for j in range(NJ):
            pltpu.async_copy(headh.at[pl.ds(base + j * CEG, CEG)],
                             headv.at[b, j], isem[b])

    def wait_idx(b):
        pltpu.make_async_copy(tailh.at[pl.ds(0, ECHUNK)], tailv.at[b],
                              isem[b]).wait()
        pltpu.make_async_copy(eth.at[pl.ds(0, ECHUNK)], etv.at[b],
                              isem[b]).wait()
        for j in range(NJ):
            pltpu.make_async_copy(headh.at[pl.ds(0, CEG)], headv.at[b, j],
                                  isem[b]).wait()

    def wait_scatter(bg):
        for j in range(NJ):
            pltpu.make_async_copy(tab.at[pl.ds(0, CEG)],
                                  g.at[bg, pl.ds(j * CEG, CEG)],
                                  ssem[bg]).wait()
        if with_cnt:
            pltpu.make_async_copy(zc.at[pl.ds(0, CEG)], onesb,
                                  ssem[bg]).wait()

    def do_chunk(k, off, first_pair=False):
        b, bg = off % 4, off % 2
        wait_idx(b)
        for v in range(ECHUNK // L):
            t = tailv[b, pl.ds(v * L, L)]
            e = etv[b, pl.ds(v * L, L)]
            fidx[b, v // SEG, pl.ds((v % SEG) * L, L)] = \
                (e - 1) * NENT + t
        if not first_pair:
            wait_scatter(bg)
        if isinstance(k, int):
            if k + 2 < NCH_E:
                start_idx(k + 2, (off + 2) % 4)
        else:
            @pl.when(k + 2 < NCH_E)
            def _():
                start_idx(k + 2, (off + 2) % 4)
        for j in range(NJ):
            pltpu.async_copy(tab.at[fidx.at[b, j]],
                             g.at[bg, pl.ds(j * CEG, CEG)], gsem)
        for j in range(NJ):
            pltpu.make_async_copy(tab.at[pl.ds(0, CEG)],
                                  g.at[bg, pl.ds(j * CEG, CEG)],
                                  gsem).wait()
        for j in range(NJ):
            pltpu.async_copy(g.at[bg, pl.ds(j * CEG, CEG)],
                             acc.at[headv.at[b, j]], ssem[bg], add=True)
        if with_cnt:
            pltpu.async_copy(onesb, cacc.at[headv.at[b, 0]], ssem[bg],
                             add=True)

    start_idx(0, 0)
    start_idx(1, 1)
    plsc.subcore_barrier()
    do_chunk(0, 0, first_pair=True)
    do_chunk(1, 1, first_pair=True)

    def body(i, carry):
        for off in range(4):
            do_chunk(4 * i + 2 + off, 2 + off)
        return carry

    lax.fori_loop(0, (NCH_E - 3) // 4, body, 0)
    for k in range(2 + 4 * ((NCH_E - 3) // 4), NCH_E):
        do_chunk(k, k % 4)
    wait_scatter((NCH_E - 2) % 2)
    wait_scatter((NCH_E - 1) % 2)
    plsc.subcore_barrier()

    def dump(nrows):
        pltpu.sync_copy(acc.at[pl.ds(r0, nrows)],
                        ea_out.at[cid, pl.ds(r0, nrows)])
        if with_cnt:
            pltpu.sync_copy(cacc.at[pl.ds(r0, nrows)],
                            cnt_out.at[cid, pl.ds(r0, nrows)])

    @pl.when(sid < NS - 1)
    def _():
        dump(RPTE)

    @pl.when(sid == NS - 1)
    def _():
        dump(TAILR)


def _user_body(ent, colh, rowh, valh, zh, ua_out,
               colv, rowv, valv, g,
               isem0, isem1, isem2, isem3, gsem, ssem0, ssem1, acc):
    cid = lax.axis_index("c")
    sid = lax.axis_index("s")
    w = cid * NS + sid
    r0 = sid * RPT
    isem = (isem0, isem1, isem2, isem3)
    ssem = (ssem0, ssem1)

    pltpu.sync_copy(zh, acc.at[pl.ds(r0, RPT)])

    def start_idx(k, b):
        base = w * IPW + k * CEU
        pltpu.async_copy(colh.at[pl.ds(base, CEU)], colv.at[b], isem[b])
        pltpu.async_copy(rowh.at[pl.ds(base, CEU)], rowv.at[b], isem[b])
        pltpu.async_copy(valh.at[pl.ds(base, CEU)], valv.at[b], isem[b])

    def wait_idx(b):
        pltpu.make_async_copy(colh.at[pl.ds(0, CEU)], colv.at[b],
                              isem[b]).wait()
        pltpu.make_async_copy(rowh.at[pl.ds(0, CEU)], rowv.at[b],
                              isem[b]).wait()
        pltpu.make_async_copy(valh.at[pl.ds(0, CEU)], valv.at[b],
                              isem[b]).wait()

    def wait_scatter(bg):
        pltpu.make_async_copy(ent.at[pl.ds(0, CEU)], g.at[bg],
                              ssem[bg]).wait()

    def do_chunk(k, off, first_pair=False):
        b, bg = off % 4, off % 2
        wait_idx(b)
        if not first_pair:
            wait_scatter(bg)
        if isinstance(k, int):
            if k + 2 < NCH_I:
                start_idx(k + 2, (off + 2) % 4)
        else:
            @pl.when(k + 2 < NCH_I)
            def _():
                start_idx(k + 2, (off + 2) % 4)
        pltpu.async_copy(ent.at[colv.at[b]], g.at[bg], gsem)
        pltpu.make_async_copy(ent.at[pl.ds(0, CEU)], g.at[bg], gsem).wait()
        bsp = jnp.full((L,), b, jnp.int32)

        def edge(e4, c2):
            for u in range(4):
                e = e4 * 4 + u
                esp = jnp.full((L,), e, jnp.int32)
                vsp = plsc.load_gather(valv, [bsp, esp])
                for s in range(SEG):
                    gv = g[bg, e, pl.ds(s * L, L)]
                    g[bg, e, pl.ds(s * L, L)] = gv * vsp
            return c2

        lax.fori_loop(0, CEU // 4, edge, 0)
        pltpu.async_copy(g.at[bg], acc.at[rowv.at[b]], ssem[bg], add=True)

    start_idx(0, 0)
    start_idx(1, 1)
    plsc.subcore_barrier()
    do_chunk(0, 0, first_pair=True)
    do_chunk(1, 1, first_pair=True)

    def body(i, carry):
        for off in range(4):
            do_chunk(4 * i + 2 + off, 2 + off)
        return carry

    lax.fori_loop(0, (NCH_I - 3) // 4, body, 0)
    for k in range(2 + 4 * ((NCH_I - 3) // 4), NCH_I):
        do_chunk(k, k % 4)
    wait_scatter((NCH_I - 2) % 2)
    wait_scatter((NCH_I - 1) % 2)
    plsc.subcore_barrier()
    pltpu.sync_copy(acc.at[pl.ds(r0, RPT)], ua_out.at[cid, pl.ds(r0, RPT)])


def _table_body(e_ref, w_ref, tab_ref):
    tab_ref[...] = e_ref[...][None, :, :] * w_ref[...][:, None, :]


def _table_call(ecur, w):
    n = NENT // BLK
    return pl.pallas_call(
        _table_body,
        grid=(n,),
        in_specs=[
            pl.BlockSpec((BLK, CH), lambda i: (i, 0)),
            pl.BlockSpec((NRELM1, CH), lambda i: (0, 0)),
        ],
        out_specs=pl.BlockSpec((NRELM1, BLK, CH), lambda i: (0, i, 0)),
        out_shape=jax.ShapeDtypeStruct((NRELM1, NENT, CH), jnp.float32),
    )(ecur, w)


def _l2n(x):
    sq = jnp.sum(x * x, axis=1, keepdims=True)
    return x * lax.rsqrt(jnp.maximum(sq, 1e-12))


BLK = 1000


def _update_common(ea_ref, cnt_ref, ua_ref, u_ref, eres_ref, ures_ref,
                   lat_ref, dis_ref, enew_ref, unew_ref, ereso_ref,
                   ureso_ref):
    ea = ea_ref[0] + ea_ref[1]
    cnt = cnt_ref[0, :, 0:1] + cnt_ref[1, :, 0:1]
    agg = jnp.where(cnt != 0.0, ea / jnp.where(cnt != 0.0, cnt, 1.0), 0.0)
    enew = _l2n(agg)
    enew_ref[...] = enew
    ereso_ref[...] = eres_ref[...] + enew

    u = u_ref[...]
    logits = lax.dot_general(u, lat_ref[...], (((1,), (1,)), ((), ())))
    m = jnp.max(logits, axis=1, keepdims=True)
    ex = jnp.exp(logits - m)
    score = ex / jnp.sum(ex, axis=1, keepdims=True)
    fac = 1.0 + lax.dot_general(score, dis_ref[...], (((1,), (0,)), ((), ())))
    ua = (ua_ref[0] + ua_ref[1]) * fac
    unew = _l2n(ua)
    unew_ref[...] = unew
    ureso_ref[...] = ures_ref[...] + unew
    return enew


def _update_body(ea_ref, cnt_ref, ua_ref, u_ref, eres_ref, ures_ref,
                 lat_ref, dis_ref, enew_ref, unew_ref, ereso_ref, ureso_ref):
    _update_common(ea_ref, cnt_ref, ua_ref, u_ref, eres_ref, ures_ref,
                   lat_ref, dis_ref, enew_ref, unew_ref, ereso_ref, ureso_ref)


def _update_body_tab(ea_ref, cnt_ref, ua_ref, u_ref, eres_ref, ures_ref,
                     lat_ref, dis_ref, w_ref, enew_ref, unew_ref, ereso_ref,
                     ureso_ref, tab_ref):
    enew = _update_common(ea_ref, cnt_ref, ua_ref, u_ref, eres_ref, ures_ref,
                          lat_ref, dis_ref, enew_ref, unew_ref, ereso_ref,
                          ureso_ref)
    tab_ref[...] = enew[None, :, :] * w_ref[...][:, None, :]


def _update_call(ea, cnt, ua, u, eres, ures, lat, dis, w=None):
    n = NENT // BLK
    in_specs = [
        pl.BlockSpec((NC, BLK, CH), lambda i: (0, i, 0)),
        pl.BlockSpec((NC, BLK, L), lambda i: (0, i, 0)),
        pl.BlockSpec((NC, BLK, CH), lambda i: (0, i, 0)),
        pl.BlockSpec((BLK, CH), lambda i: (i, 0)),
        pl.BlockSpec((BLK, CH), lambda i: (i, 0)),
        pl.BlockSpec((BLK, CH), lambda i: (i, 0)),
        pl.BlockSpec((NFACT, CH), lambda i: (0, 0)),
        pl.BlockSpec((NFACT, CH), lambda i: (0, 0)),
    ]
    out_specs = [
        pl.BlockSpec((BLK, CH), lambda i: (i, 0)),
        pl.BlockSpec((BLK, CH), lambda i: (i, 0)),
        pl.BlockSpec((BLK, CH), lambda i: (i, 0)),
        pl.BlockSpec((BLK, CH), lambda i: (i, 0)),
    ]
    out_shape = [
        jax.ShapeDtypeStruct((NENT, CH), jnp.float32),
        jax.ShapeDtypeStruct((NUSERS, CH), jnp.float32),
        jax.ShapeDtypeStruct((NENT, CH), jnp.float32),
        jax.ShapeDtypeStruct((NUSERS, CH), jnp.float32),
    ]
    args = (ea, cnt, ua, u, eres, ures, lat, dis)
    if w is None:
        return pl.pallas_call(
            _update_body, grid=(n,), in_specs=in_specs,
            out_specs=out_specs, out_shape=out_shape)(*args)
    in_specs.append(pl.BlockSpec((NRELM1, CH), lambda i: (0, 0)))
    out_specs.append(pl.BlockSpec((NRELM1, BLK, CH), lambda i: (0, i, 0)))
    out_shape.append(jax.ShapeDtypeStruct((NRELM1, NENT, CH), jnp.float32))
    return pl.pallas_call(
        _update_body_tab, grid=(n,), in_specs=in_specs,
        out_specs=out_specs, out_shape=out_shape)(*args, w)


def _cor_body(att_ref, w_ref, cor_ref, dis_ref):
    att = att_ref[...]
    nrm = jnp.sqrt(jnp.sum(att * att, axis=1, keepdims=True))
    n = att / nrm
    gram = lax.dot_general(n, n, (((1,), (1,)), ((), ())))
    g2 = gram * gram
    ii = lax.broadcasted_iota(jnp.int32, (NFACT, NFACT), 0)
    jj = lax.broadcasted_iota(jnp.int32, (NFACT, NFACT), 1)
    cor_ref[...] = jnp.sum(jnp.where(ii < jj, g2, 0.0)) * jnp.ones((1, 1),
                                                                   jnp.float32)
    m = jnp.max(att, axis=1, keepdims=True)
    ex = jnp.exp(att - m)
    sm = ex / jnp.sum(ex, axis=1, keepdims=True)
    dis_ref[...] = lax.dot_general(sm, w_ref[...], (((1,), (0,)), ((), ())))


def _cor_call(att, w):
    return pl.pallas_call(
        _cor_body,
        out_shape=[
            jax.ShapeDtypeStruct((1, 1), jnp.float32),
            jax.ShapeDtypeStruct((NFACT, CH), jnp.float32),
        ],
    )(att, w)


def kernel(user_emb, entity_emb, latent_emb, edge_index, edge_type,
           interact_rows, interact_cols, interact_vals, weight,
           disen_weight_att):
    nedge = edge_type.shape[0]
    epw0 = nedge // NW
    epad = EPW - epw0

    def sprd(x, padrow):
        return jnp.concatenate(
            [x.reshape(NW, epw0),
             jnp.broadcast_to(padrow, (NW, epad)).astype(jnp.int32)],
            axis=1).reshape(-1)

    head = sprd(edge_index[0].astype(jnp.int32),
                NENT + (jnp.arange(epad, dtype=jnp.int32) % L))
    tail = sprd(edge_index[1].astype(jnp.int32),
                jnp.zeros((epad,), jnp.int32))
    et = sprd(edge_type.astype(jnp.int32), jnp.ones((epad,), jnp.int32))

    nnz = interact_rows.shape[0]
    ipw0 = nnz // NW
    ipad = IPW - ipw0

    def isprd(x, padrow):
        return jnp.concatenate(
            [x.reshape(NW, ipw0),
             jnp.broadcast_to(padrow, (NW, ipad)).astype(x.dtype)],
            axis=1).reshape(-1)

    irows = isprd(interact_rows.astype(jnp.int32),
                  jnp.arange(ipad, dtype=jnp.int32))
    icols = isprd(interact_cols.astype(jnp.int32),
                  jnp.zeros((ipad,), jnp.int32))
    ivals = isprd(interact_vals, jnp.zeros((ipad,), jnp.float32))
    z128e = jnp.zeros((RPTE, CH), jnp.float32)
    z16 = jnp.zeros((RPTE, L), jnp.float32)
    z128u = jnp.zeros((RPT, CH), jnp.float32)

    cor2d, disen = _cor_call(disen_weight_att, weight)
    edge_cnt, edge_nc, user_call = _sc_calls()

    tab1 = _table_call(entity_emb, weight).reshape(NTAB, CH)
    ea1, cnt1 = edge_cnt(tab1, tail, head, et, z128e, z16)
    ua1 = user_call(entity_emb, icols, irows, ivals, z128u)
    e1, u1, eres1, ures1, tab2_3d = _update_call(
        ea1, cnt1, ua1, user_emb, entity_emb, user_emb, latent_emb, disen,
        w=weight)

    tab2 = tab2_3d.reshape(NTAB, CH)
    ea2 = edge_nc(tab2, tail, head, et, z128e)
    ua2 = user_call(e1, icols, irows, ivals, z128u)
    _, _, eres2, ures2 = _update_call(
        ea2, cnt1, ua2, u1, eres1, ures1, latent_emb, disen)
    return eres2, ures2, cor2d[0, 0]


# speedup vs baseline: 1.0007x; 1.0007x over previous
import functools

import jax
import jax.numpy as jnp
from jax import lax
from jax.experimental import pallas as pl
from jax.experimental.pallas import tpu as pltpu
from jax.experimental.pallas import tpu_sc as plsc

NUSERS = 10000
NENT = 10000
CH = 128
NEDGE = 320000
NFACT = 4
NRELM1 = 8
NC = 2
NS = 16
NW = NC * NS
L = 16

CEG = 128
ECHUNK = 128
EPW = 10112
EPAD = EPW * NW
NCH_E = EPW // ECHUNK
CEU = 128
IPW = 6272
NNZ_PAD2 = IPW * NW
NCH_I = IPW // CEU
RPT = NENT // NS
SEG = CH // L
NTAB = NRELM1 * NENT
ACC_E = 10016
RPTE = ACC_E // NS
TAILR = NENT - 15 * RPTE

@functools.cache
def _sc_calls():
    mesh = plsc.VectorSubcoreMesh(core_axis_name="c", subcore_axis_name="s",
                                  num_cores=NC, num_subcores=NS)
    sems = [pltpu.SemaphoreType.DMA] * 7
    idx_scratch = [
        pltpu.VMEM((4, ECHUNK), jnp.int32),
        pltpu.VMEM((4, ECHUNK), jnp.int32),
        pltpu.VMEM((4, NJ, CEG), jnp.int32),
        pltpu.VMEM((4, NJ, CEG), jnp.int32),
        pltpu.VMEM((2, ECHUNK, CH), jnp.float32),
    ]
    edge_call_cnt = functools.partial(
        pl.kernel,
        out_type=[
            jax.ShapeDtypeStruct((NC, NENT, CH), jnp.float32),
            jax.ShapeDtypeStruct((NC, NENT, L), jnp.float32),
        ],
        mesh=mesh,
        compiler_params=pltpu.CompilerParams(use_tc_tiling_on_sc=False, needs_layout_passes=False),
        scratch_types=idx_scratch + [pltpu.VMEM((CEG, L), jnp.float32)]
        + sems + [
            pltpu.VMEM_SHARED((ACC_E, CH), jnp.float32),
            pltpu.VMEM_SHARED((ACC_E, L), jnp.float32),
        ],
    )(_edge_body_cnt)
    edge_call_nc = functools.partial(
        pl.kernel,
        out_type=jax.ShapeDtypeStruct((NC, NENT, CH), jnp.float32),
        mesh=mesh,
        compiler_params=pltpu.CompilerParams(use_tc_tiling_on_sc=False, needs_layout_passes=False),
        scratch_types=idx_scratch + sems
        + [pltpu.VMEM_SHARED((ACC_E, CH), jnp.float32)],
    )(_edge_body_nc)
    user_call = functools.partial(
        pl.kernel,
        out_type=jax.ShapeDtypeStruct((NC, NUSERS, CH), jnp.float32),
        mesh=mesh,
        compiler_params=pltpu.CompilerParams(use_tc_tiling_on_sc=False, needs_layout_passes=False),
        scratch_types=[
            pltpu.VMEM((4, CEU), jnp.int32),
            pltpu.VMEM((4, CEU), jnp.int32),
            pltpu.VMEM((4, CEU), jnp.float32),
            pltpu.VMEM((2, CEU, CH), jnp.float32),
            pltpu.SemaphoreType.DMA,
            pltpu.SemaphoreType.DMA,
            pltpu.SemaphoreType.DMA,
            pltpu.SemaphoreType.DMA,
            pltpu.SemaphoreType.DMA,
            pltpu.SemaphoreType.DMA,
            pltpu.SemaphoreType.DMA,
            pltpu.VMEM_SHARED((NUSERS, CH), jnp.float32),
        ],
    )(_user_body)
    return edge_call_cnt, edge_call_nc, user_call


NJ = ECHUNK // CEG


def _edge_body_cnt(tab, tailh, headh, eth, zh, zc, ea, cnt,
                   tailv, etv, headv, fidx, g, onesb,
                   i0, i1, i2, i3, gs, s0, s1, acc, cacc):
    _edge_impl(tab, tailh, headh, eth, zh, ea, tailv, etv, headv, fidx, g,
               i0, i1, i2, i3, gs, s0, s1, acc,
               zc=zc, cnt_out=cnt, onesb=onesb, cacc=cacc)


def _edge_body_nc(tab, tailh, headh, eth, zh, ea,
                  tailv, etv, headv, fidx, g,
                  i0, i1, i2, i3, gs, s0, s1, acc):
    _edge_impl(tab, tailh, headh, eth, zh, ea, tailv, etv, headv, fidx, g,
               i0, i1, i2, i3, gs, s0, s1, acc)


def _edge_impl(tab, tailh, headh, eth, zh, ea_out,
               tailv, etv, headv, fidx, g,
               isem0, isem1, isem2, isem3, gsem, ssem0, ssem1, acc,
               zc=None, cnt_out=None, onesb=None, cacc=None):
    with_cnt = cacc is not None
    cid = lax.axis_index("c")
    sid = lax.axis_index("s")
    w = cid * NS + sid
    r0 = sid * RPTE
    isem = (isem0, isem1, isem2, isem3)
    ssem = (ssem0, ssem1)

    pltpu.sync_copy(zh, acc.at[pl.ds(r0, RPTE)])
    if with_cnt:
        pltpu.sync_copy(zc, cacc.at[pl.ds(r0, RPTE)])
        one = jnp.ones((L,), jnp.float32)
        for e in range(CEG):
            onesb[e, :] = one

    def start_idx(k, b):
        base = w * EPW + k * ECHUNK
        pltpu.async_copy(tailh.at[pl.ds(base, ECHUNK)], tailv.at[b], isem[b])
        pltpu.async_copy(eth.at[pl.ds(base, ECHUNK)], etv.at[b], isem[b])
        for j in range(NJ):
            pltpu.async_copy(headh.at[pl.ds(base + j * CEG, CEG)],
                             headv.at[b, j], isem[b])

    def wait_idx(b):
        pltpu.make_async_copy(tailh.at[pl.ds(0, ECHUNK)], tailv.at[b],
                              isem[b]).wait()
        pltpu.make_async_copy(eth.at[pl.ds(0, ECHUNK)], etv.at[b],
                              isem[b]).wait()
        for j in range(NJ):
            pltpu.make_async_copy(headh.at[pl.ds(0, CEG)], headv.at[b, j],
                                  isem[b]).wait()

    def wait_scatter(bg):
        for j in range(NJ):
            pltpu.make_async_copy(tab.at[pl.ds(0, CEG)],
                                  g.at[bg, pl.ds(j * CEG, CEG)],
                                  ssem[bg]).wait()
        if with_cnt:
            pltpu.make_async_copy(zc.at[pl.ds(0, CEG)], onesb,
                                  ssem[bg]).wait()

    def do_chunk(k, off, first_pair=False):
        b, bg = off % 4, off % 2
        wait_idx(b)
        for v in range(ECHUNK // L):
            t = tailv[b, pl.ds(v * L, L)]
            e = etv[b, pl.ds(v * L, L)]
            fidx[b, v // SEG, pl.ds((v % SEG) * L, L)] = \
                (e - 1) * NENT + t
        if not first_pair:
            wait_scatter(bg)
        if isinstance(k, int):
            if k + 2 < NCH_E:
                start_idx(k + 2, (off + 2) % 4)
        else:
            @pl.when(k + 2 < NCH_E)
            def _():
                start_idx(k + 2, (off + 2) % 4)
        for j in range(NJ):
            pltpu.async_copy(tab.at[fidx.at[b, j]],
                             g.at[bg, pl.ds(j * CEG, CEG)], gsem)
        for j in range(NJ):
            pltpu.make_async_copy(tab.at[pl.ds(0, CEG)],
                                  g.at[bg, pl.ds(j * CEG, CEG)],
                                  gsem).wait()
        for j in range(NJ):
            pltpu.async_copy(g.at[bg, pl.ds(j * CEG, CEG)],
                             acc.at[headv.at[b, j]], ssem[bg], add=True)
        if with_cnt:
            pltpu.async_copy(onesb, cacc.at[headv.at[b, 0]], ssem[bg],
                             add=True)

    start_idx(0, 0)
    start_idx(1, 1)
    plsc.subcore_barrier()
    do_chunk(0, 0, first_pair=True)
    do_chunk(1, 1, first_pair=True)

    def body(i, carry):
        for off in range(4):
            do_chunk(4 * i + 2 + off, 2 + off)
        return carry

    lax.fori_loop(0, (NCH_E - 3) // 4, body, 0)
    for k in range(2 + 4 * ((NCH_E - 3) // 4), NCH_E):
        do_chunk(k, k % 4)
    wait_scatter((NCH_E - 2) % 2)
    wait_scatter((NCH_E - 1) % 2)
    plsc.subcore_barrier()

    def dump(nrows):
        pltpu.sync_copy(acc.at[pl.ds(r0, nrows)],
                        ea_out.at[cid, pl.ds(r0, nrows)])
        if with_cnt:
            pltpu.sync_copy(cacc.at[pl.ds(r0, nrows)],
                            cnt_out.at[cid, pl.ds(r0, nrows)])

    @pl.when(sid < NS - 1)
    def _():
        dump(RPTE)

    @pl.when(sid == NS - 1)
    def _():
        dump(TAILR)


def _user_body(ent, colh, rowh, valh, zh, ua_out,
               colv, rowv, valv, g,
               isem0, isem1, isem2, isem3, gsem, ssem0, ssem1, acc):
    cid = lax.axis_index("c")
    sid = lax.axis_index("s")
    w = cid * NS + sid
    r0 = sid * RPT
    isem = (isem0, isem1, isem2, isem3)
    ssem = (ssem0, ssem1)

    pltpu.sync_copy(zh, acc.at[pl.ds(r0, RPT)])

    def start_idx(k, b):
        base = w * IPW + k * CEU
        pltpu.async_copy(colh.at[pl.ds(base, CEU)], colv.at[b], isem[b])
        pltpu.async_copy(rowh.at[pl.ds(base, CEU)], rowv.at[b], isem[b])
        pltpu.async_copy(valh.at[pl.ds(base, CEU)], valv.at[b], isem[b])

    def wait_idx(b):
        pltpu.make_async_copy(colh.at[pl.ds(0, CEU)], colv.at[b],
                              isem[b]).wait()
        pltpu.make_async_copy(rowh.at[pl.ds(0, CEU)], rowv.at[b],
                              isem[b]).wait()
        pltpu.make_async_copy(valh.at[pl.ds(0, CEU)], valv.at[b],
                              isem[b]).wait()

    def wait_scatter(bg):
        pltpu.make_async_copy(ent.at[pl.ds(0, CEU)], g.at[bg],
                              ssem[bg]).wait()

    def do_chunk(k, off, first_pair=False):
        b, bg = off % 4, off % 2
        wait_idx(b)
        if not first_pair:
            wait_scatter(bg)
        if isinstance(k, int):
            if k + 2 < NCH_I:
                start_idx(k + 2, (off + 2) % 4)
        else:
            @pl.when(k + 2 < NCH_I)
            def _():
                start_idx(k + 2, (off + 2) % 4)
        pltpu.async_copy(ent.at[colv.at[b]], g.at[bg], gsem)
        pltpu.make_async_copy(ent.at[pl.ds(0, CEU)], g.at[bg], gsem).wait()
        bsp = jnp.full((L,), b, jnp.int32)

        def edge(e4, c2):
            for u in range(4):
                e = e4 * 4 + u
                esp = jnp.full((L,), e, jnp.int32)
                vsp = plsc.load_gather(valv, [bsp, esp])
                for s in range(SEG):
                    gv = g[bg, e, pl.ds(s * L, L)]
                    g[bg, e, pl.ds(s * L, L)] = gv * vsp
            return c2

        lax.fori_loop(0, CEU // 4, edge, 0)
        pltpu.async_copy(g.at[bg], acc.at[rowv.at[b]], ssem[bg], add=True)

    start_idx(0, 0)
    start_idx(1, 1)
    plsc.subcore_barrier()
    do_chunk(0, 0, first_pair=True)
    do_chunk(1, 1, first_pair=True)

    def body(i, carry):
        for off in range(4):
            do_chunk(4 * i + 2 + off, 2 + off)
        return carry

    lax.fori_loop(0, (NCH_I - 3) // 4, body, 0)
    for k in range(2 + 4 * ((NCH_I - 3) // 4), NCH_I):
        do_chunk(k, k % 4)
    wait_scatter((NCH_I - 2) % 2)
    wait_scatter((NCH_I - 1) % 2)
    plsc.subcore_barrier()
    pltpu.sync_copy(acc.at[pl.ds(r0, RPT)], ua_out.at[cid, pl.ds(r0, RPT)])


def _table_body(e_ref, w_ref, tab_ref):
    tab_ref[...] = e_ref[...][None, :, :] * w_ref[...][:, None, :]


def _table_call(ecur, w):
    n = NENT // BLK
    return pl.pallas_call(
        _table_body,
        grid=(n,),
        in_specs=[
            pl.BlockSpec((BLK, CH), lambda i: (i, 0)),
            pl.BlockSpec((NRELM1, CH), lambda i: (0, 0)),
        ],
        out_specs=pl.BlockSpec((NRELM1, BLK, CH), lambda i: (0, i, 0)),
        out_shape=jax.ShapeDtypeStruct((NRELM1, NENT, CH), jnp.float32),
    )(ecur, w)


def _l2n(x):
    sq = jnp.sum(x * x, axis=1, keepdims=True)
    return x * lax.rsqrt(jnp.maximum(sq, 1e-12))


BLK = 1000


def _update_common(ea_ref, cnt_ref, ua_ref, u_ref, eres_ref, ures_ref,
                   lat_ref, dis_ref, enew_ref, unew_ref, ereso_ref,
                   ureso_ref):
    ea = ea_ref[0] + ea_ref[1]
    cnt = cnt_ref[0, :, 0:1] + cnt_ref[1, :, 0:1]
    agg = jnp.where(cnt != 0.0, ea / jnp.where(cnt != 0.0, cnt, 1.0), 0.0)
    enew = _l2n(agg)
    enew_ref[...] = enew
    ereso_ref[...] = eres_ref[...] + enew

    u = u_ref[...]
    logits = lax.dot_general(u, lat_ref[...], (((1,), (1,)), ((), ())))
    m = jnp.max(logits, axis=1, keepdims=True)
    ex = jnp.exp(logits - m)
    score = ex / jnp.sum(ex, axis=1, keepdims=True)
    fac = 1.0 + lax.dot_general(score, dis_ref[...], (((1,), (0,)), ((), ())))
    ua = (ua_ref[0] + ua_ref[1]) * fac
    unew = _l2n(ua)
    unew_ref[...] = unew
    ureso_ref[...] = ures_ref[...] + unew
    return enew


def _update_body(ea_ref, cnt_ref, ua_ref, u_ref, eres_ref, ures_ref,
                 lat_ref, dis_ref, enew_ref, unew_ref, ereso_ref, ureso_ref):
    _update_common(ea_ref, cnt_ref, ua_ref, u_ref, eres_ref, ures_ref,
                   lat_ref, dis_ref, enew_ref, unew_ref, ereso_ref, ureso_ref)


def _update_body_tab(ea_ref, cnt_ref, ua_ref, u_ref, eres_ref, ures_ref,
                     lat_ref, dis_ref, w_ref, enew_ref, unew_ref, ereso_ref,
                     ureso_ref, tab_ref):
    enew = _update_common(ea_ref, cnt_ref, ua_ref, u_ref, eres_ref, ures_ref,
                          lat_ref, dis_ref, enew_ref, unew_ref, ereso_ref,
                          ureso_ref)
    tab_ref[...] = enew[None, :, :] * w_ref[...][:, None, :]


def _update_call(ea, cnt, ua, u, eres, ures, lat, dis, w=None):
    n = NENT // BLK
    in_specs = [
        pl.BlockSpec((NC, BLK, CH), lambda i: (0, i, 0)),
        pl.BlockSpec((NC, BLK, L), lambda i: (0, i, 0)),
        pl.BlockSpec((NC, BLK, CH), lambda i: (0, i, 0)),
        pl.BlockSpec((BLK, CH), lambda i: (i, 0)),
        pl.BlockSpec((BLK, CH), lambda i: (i, 0)),
        pl.BlockSpec((BLK, CH), lambda i: (i, 0)),
        pl.BlockSpec((NFACT, CH), lambda i: (0, 0)),
        pl.BlockSpec((NFACT, CH), lambda i: (0, 0)),
    ]
    out_specs = [
        pl.BlockSpec((BLK, CH), lambda i: (i, 0)),
        pl.BlockSpec((BLK, CH), lambda i: (i, 0)),
        pl.BlockSpec((BLK, CH), lambda i: (i, 0)),
        pl.BlockSpec((BLK, CH), lambda i: (i, 0)),
    ]
    out_shape = [
        jax.ShapeDtypeStruct((NENT, CH), jnp.float32),
        jax.ShapeDtypeStruct((NUSERS, CH), jnp.float32),
        jax.ShapeDtypeStruct((NENT, CH), jnp.float32),
        jax.ShapeDtypeStruct((NUSERS, CH), jnp.float32),
    ]
    args = (ea, cnt, ua, u, eres, ures, lat, dis)
    if w is None:
        return pl.pallas_call(
            _update_body, grid=(n,), in_specs=in_specs,
            out_specs=out_specs, out_shape=out_shape)(*args)
    in_specs.append(pl.BlockSpec((NRELM1, CH), lambda i: (0, 0)))
    out_specs.append(pl.BlockSpec((NRELM1, BLK, CH), lambda i: (0, i, 0)))
    out_shape.append(jax.ShapeDtypeStruct((NRELM1, NENT, CH), jnp.float32))
    return pl.pallas_call(
        _update_body_tab, grid=(n,), in_specs=in_specs,
        out_specs=out_specs, out_shape=out_shape)(*args, w)


def _cor_body(att_ref, w_ref, cor_ref, dis_ref):
    att = att_ref[...]
    nrm = jnp.sqrt(jnp.sum(att * att, axis=1, keepdims=True))
    n = att / nrm
    gram = lax.dot_general(n, n, (((1,), (1,)), ((), ())))
    g2 = gram * gram
    ii = lax.broadcasted_iota(jnp.int32, (NFACT, NFACT), 0)
    jj = lax.broadcasted_iota(jnp.int32, (NFACT, NFACT), 1)
    cor_ref[...] = jnp.sum(jnp.where(ii < jj, g2, 0.0)) * jnp.ones((1, 1),
                                                                   jnp.float32)
    m = jnp.max(att, axis=1, keepdims=True)
    ex = jnp.exp(att - m)
    sm = ex / jnp.sum(ex, axis=1, keepdims=True)
    dis_ref[...] = lax.dot_general(sm, w_ref[...], (((1,), (0,)), ((), ())))


def _cor_call(att, w):
    return pl.pallas_call(
        _cor_body,
        out_shape=[
            jax.ShapeDtypeStruct((1, 1), jnp.float32),
            jax.ShapeDtypeStruct((NFACT, CH), jnp.float32),
        ],
    )(att, w)


def kernel(user_emb, entity_emb, latent_emb, edge_index, edge_type,
           interact_rows, interact_cols, interact_vals, weight,
           disen_weight_att):
    nedge = edge_type.shape[0]
    epw0 = nedge // NW
    epad = EPW - epw0

    def sprd(x, padrow):
        return jnp.concatenate(
            [x.reshape(NW, epw0),
             jnp.broadcast_to(padrow, (NW, epad)).astype(jnp.int32)],
            axis=1).reshape(-1)

    head = sprd(edge_index[0].astype(jnp.int32),
                NENT + (jnp.arange(epad, dtype=jnp.int32) % L))
    tail = sprd(edge_index[1].astype(jnp.int32),
                jnp.zeros((epad,), jnp.int32))
    et = sprd(edge_type.astype(jnp.int32), jnp.ones((epad,), jnp.int32))

    nnz = interact_rows.shape[0]
    ipw0 = nnz // NW
    ipad = IPW - ipw0

    def isprd(x, padrow):
        return jnp.concatenate(
            [x.reshape(NW, ipw0),
             jnp.broadcast_to(padrow, (NW, ipad)).astype(x.dtype)],
            axis=1).reshape(-1)

    irows = isprd(interact_rows.astype(jnp.int32),
                  jnp.arange(ipad, dtype=jnp.int32))
    icols = isprd(interact_cols.astype(jnp.int32),
                  jnp.zeros((ipad,), jnp.int32))
    ivals = isprd(interact_vals, jnp.zeros((ipad,), jnp.float32))
    z128e = jnp.zeros((RPTE, CH), jnp.float32)
    z16 = jnp.zeros((RPTE, L), jnp.float32)
    z128u = jnp.zeros((RPT, CH), jnp.float32)

    cor2d, disen = _cor_call(disen_weight_att, weight)
    edge_cnt, edge_nc, user_call = _sc_calls()

    tab1 = _table_call(entity_emb, weight).reshape(NTAB, CH)
    ea1, cnt1 = edge_cnt(tab1, tail, head, et, z128e, z16)
    ua1 = user_call(entity_emb, icols, irows, ivals, z128u)
    e1, u1, eres1, ures1, tab2_3d = _update_call(
        ea1, cnt1, ua1, user_emb, entity_emb, user_emb, latent_emb, disen,
        w=weight)

    tab2 = tab2_3d.reshape(NTAB, CH)
    ea2 = edge_nc(tab2, tail, head, et, z128e)
    ua2 = user_call(e1, icols, irows, ivals, z128u)
    _, _, eres2, ures2 = _update_call(
        ea2, cnt1, ua2, u1, eres1, ures1, latent_emb, disen)
    return eres2, ures2, cor2d[0, 0]
